# baseline (speedup 1.0000x reference)
import functools

import jax
import jax.numpy as jnp
from jax import lax
from jax.experimental import pallas as pl
from jax.experimental.pallas import tpu as pltpu
from jax.experimental.pallas import tpu_sc as plsc

_T = 512


def _probs_body(enc_ref, wq_ref, wk_ref, out_ref, key_ref, kprev_ref):
    t = pl.program_id(1)
    eb = enc_ref[0]
    T, d = eb.shape
    rows = lax.broadcasted_iota(jnp.int32, (T, 1), 0)
    dn = (((1,), (1,)), ((), ()))
    nchunk = 4
    cs = d // nchunk
    nq = jnp.zeros((T, 1), jnp.float32)
    nk = jnp.zeros((T, 1), jnp.float32)
    dots = jnp.zeros((T, 1), jnp.float32)
    for j in range(nchunk):
        sl = pl.ds(j * cs, cs)
        qm = lax.dot_general(eb, wq_ref[sl, :], dn,
                             precision=lax.Precision.DEFAULT,
                             preferred_element_type=jnp.float32)
        km = lax.dot_general(eb, wk_ref[sl, :], dn,
                             precision=lax.Precision.DEFAULT,
                             preferred_element_type=jnp.float32)
        km_sh = pltpu.roll(km, 1, 0)
        km_sh = jnp.where(rows == 0, kprev_ref[0:1, sl], km_sh)
        kprev_ref[0:1, sl] = km[T - 1:, :]
        nq = nq + jnp.sum(qm * qm, axis=1, keepdims=True)
        nk = nk + jnp.sum(km_sh * km_sh, axis=1, keepdims=True)
        dots = dots + jnp.sum(qm * km_sh, axis=1, keepdims=True)
    denom = jnp.maximum(jnp.sqrt(nq), 1e-12) * jnp.maximum(jnp.sqrt(nk), 1e-12)
    p = (1.0 - dots / denom) * 0.5
    p = jnp.where(jnp.logical_and(t == 0, rows == 0), 1.0, p)
    out_ref[0] = p
    u = lax.bitcast_convert_type(p, jnp.uint32)
    m = jnp.where(p < 0.0, jnp.uint32(0xFFFFFFFF), jnp.uint32(0x80000000))
    key_ref[0] = u ^ m


def _compute_probs(enc, wq_bf, wk_bf):
    B, L, d = enc.shape
    nt = L // _T
    out, keys = pl.pallas_call(
        _probs_body,
        grid=(B, nt),
        in_specs=[
            pl.BlockSpec((1, _T, d), lambda b, t: (b, t, 0)),
            pl.BlockSpec((d, d), lambda b, t: (0, 0)),
            pl.BlockSpec((d, d), lambda b, t: (0, 0)),
        ],
        out_specs=[
            pl.BlockSpec((1, _T, 1), lambda b, t: (b * nt + t, 0, 0)),
            pl.BlockSpec((1, _T, 1), lambda b, t: (b * nt + t, 0, 0)),
        ],
        out_shape=[
            jax.ShapeDtypeStruct((B * nt, _T, 1), jnp.float32),
            jax.ShapeDtypeStruct((B * nt, _T, 1), jnp.uint32),
        ],
        scratch_shapes=[pltpu.VMEM((1, d), jnp.float32)],
    )(enc, wq_bf, wk_bf)
    return out.reshape(B, L), keys.reshape(B * L)


def _make_topk(B, L, M):
    nv = L // 16
    mesh = plsc.VectorSubcoreMesh(core_axis_name="c", subcore_axis_name="s")

    @functools.partial(
        pl.kernel,
        mesh=mesh,
        out_type=jax.ShapeDtypeStruct((B * M,), jnp.int32),
        compiler_params=pltpu.CompilerParams(needs_layout_passes=False),
        scratch_types=[
            pltpu.VMEM((L,), jnp.uint32),
            pltpu.VMEM((M,), jnp.int32),
            pltpu.VMEM((256,), jnp.int32),
        ],
    )
    def _topk(keys_hbm, out_hbm, keys_v, idx_v, hist_v):
        wid = lax.axis_index("s") * 2 + lax.axis_index("c")

        @pl.when(wid < B)
        def _():
            pltpu.sync_copy(keys_hbm.at[pl.ds(wid * L, L)], keys_v)

            iota16 = lax.broadcasted_iota(jnp.int32, (16,), 0)
            ones16 = jnp.full((16,), 1, jnp.int32)
            zero16 = jnp.zeros((16,), jnp.int32)
            fifteen = jnp.full((16,), 15, jnp.int32)
            gdn = lax.GatherDimensionNumbers(
                offset_dims=(), collapsed_slice_dims=(0,), start_index_map=(0,))

            def gat(x, idxv):
                return lax.gather(x, idxv[:, None], gdn, (1,),
                                  mode=lax.GatherScatterMode.PROMISE_IN_BOUNDS)

            def splat_last(x):
                return gat(x, fifteen)

            def rev(x):
                return lax.rev(x, (0,))

            prefix = jnp.zeros((16,), jnp.uint32)
            rem = jnp.full((16,), M, jnp.int32)
            for r in range(4):
                bsh = jnp.uint32(24 - 8 * r)

                def hz(i, c):
                    hist_v[pl.ds(i * 16, 16)] = zero16
                    return c

                lax.fori_loop(0, 16, hz, 0, unroll=4)

                def hb(i, c, _bsh=bsh, _prefix=prefix, _r=r):
                    kv = keys_v[pl.ds(i * 16, 16)]
                    bucket = ((kv >> _bsh) & jnp.uint32(255)).astype(jnp.int32)
                    if _r == 0:
                        plsc.addupdate_scatter(hist_v, [bucket], ones16)
                    else:
                        match = (kv >> (_bsh + 8)) == (_prefix >> (_bsh + 8))
                        plsc.addupdate_scatter(hist_v, [bucket], ones16,
                                               mask=match)
                    return c

                lax.fori_loop(0, nv, hb, 0, unroll=4)

                def hs(c, carry, _rem=rem):
                    cum, bfound, g = carry
                    cc = 15 - c
                    h = hist_v[pl.ds(cc * 16, 16)]
                    suf_in = rev(plsc.cumsum(rev(h)))
                    suffix = suf_in + cum
                    cond = suffix >= _rem
                    npos = plsc.all_reduce_population_count(cond)
                    hit = jnp.logical_and(npos > 0, bfound < 0)
                    l = npos - 1
                    gg = jnp.where(l >= 15, cum,
                                   gat(suffix, jnp.minimum(l + 1, fifteen)))
                    bfound = jnp.where(hit, cc * 16 + l, bfound)
                    g = jnp.where(hit, gg, g)
                    cum = gat(suffix, zero16)
                    return cum, bfound, g

                _, b, g = lax.fori_loop(
                    0, 16, hs,
                    (zero16, jnp.full((16,), -1, jnp.int32), zero16))
                prefix = prefix | (b.astype(jnp.uint32) << bsh)
                rem = rem - g

            thr = prefix
            k_eq = rem

            def sel(i, carry):
                off, eqb = carry
                kv = keys_v[pl.ds(i * 16, 16)]
                gt = kv > thr
                eq = kv == thr
                eqc = plsc.cumsum(jnp.where(eq, jnp.int32(1), jnp.int32(0)))
                keep = jnp.logical_and(eq, eqb + eqc - 1 < k_eq)
                m = jnp.logical_or(gt, keep)
                mi = jnp.where(m, jnp.int32(1), jnp.int32(0))
                incl = plsc.cumsum(mi)
                pos = off + incl - 1
                plsc.store_scatter(idx_v, [pos], iota16 + i * 16, mask=m)
                return off + splat_last(incl), eqb + splat_last(eqc)

            lax.fori_loop(0, nv, sel, (zero16, zero16), unroll=4)
            pltpu.sync_copy(idx_v, out_hbm.at[pl.ds(wid * M, M)])

    return _topk


def kernel(enc, W_q, W_k):
    B, L, d = enc.shape
    M = L // 4
    probs, keys = _compute_probs(enc, W_q, W_k)
    idx = _make_topk(B, L, M)(keys).reshape(B, M)
    return probs, idx, probs


# speedup vs baseline: 1.0142x; 1.0142x over previous
import functools

import jax
import jax.numpy as jnp
from jax import lax
from jax.experimental import pallas as pl
from jax.experimental.pallas import tpu as pltpu
from jax.experimental.pallas import tpu_sc as plsc

_T = 512


def _probs_body(enc_ref, wq_ref, wk_ref, out_ref, key_ref, kprev_ref):
    t = pl.program_id(1)
    eb = enc_ref[0]
    dn = (((1,), (1,)), ((), ()))
    qm = lax.dot_general(eb, wq_ref[...], dn, precision=lax.Precision.DEFAULT,
                         preferred_element_type=jnp.float32)
    km = lax.dot_general(eb, wk_ref[...], dn, precision=lax.Precision.DEFAULT,
                         preferred_element_type=jnp.float32)
    rows = lax.broadcasted_iota(jnp.int32, (eb.shape[0], 1), 0)
    km_sh = pltpu.roll(km, 1, 0)
    km_sh = jnp.where(rows == 0, kprev_ref[...], km_sh)
    kprev_ref[...] = km[eb.shape[0] - 1:, :]
    nq = jnp.sum(qm * qm, axis=1, keepdims=True)
    nk = jnp.sum(km_sh * km_sh, axis=1, keepdims=True)
    dots = jnp.sum(qm * km_sh, axis=1, keepdims=True)
    denom = jnp.maximum(jnp.sqrt(nq), 1e-12) * jnp.maximum(jnp.sqrt(nk), 1e-12)
    p = (1.0 - dots / denom) * 0.5
    p = jnp.where(jnp.logical_and(t == 0, rows == 0), 1.0, p)
    out_ref[0] = p
    u = lax.bitcast_convert_type(p, jnp.uint32)
    m = jnp.where(p < 0.0, jnp.uint32(0xFFFFFFFF), jnp.uint32(0x80000000))
    key_ref[0] = u ^ m


def _compute_probs(enc, wq_bf, wk_bf):
    B, L, d = enc.shape
    nt = L // _T
    out, keys = pl.pallas_call(
        _probs_body,
        grid=(B, nt),
        in_specs=[
            pl.BlockSpec((1, _T, d), lambda b, t: (b, t, 0)),
            pl.BlockSpec((d, d), lambda b, t: (0, 0)),
            pl.BlockSpec((d, d), lambda b, t: (0, 0)),
        ],
        out_specs=[
            pl.BlockSpec((1, _T, 1), lambda b, t: (b * nt + t, 0, 0)),
            pl.BlockSpec((1, _T, 1), lambda b, t: (b * nt + t, 0, 0)),
        ],
        out_shape=[
            jax.ShapeDtypeStruct((B * nt, _T, 1), jnp.float32),
            jax.ShapeDtypeStruct((B * nt, _T, 1), jnp.uint32),
        ],
        scratch_shapes=[pltpu.VMEM((1, d), jnp.float32)],
    )(enc, wq_bf, wk_bf)
    return out.reshape(B, L), keys.reshape(B * L)


def _make_topk(B, L, M):
    nv = L // 16
    mesh = plsc.VectorSubcoreMesh(core_axis_name="c", subcore_axis_name="s")

    @functools.partial(
        pl.kernel,
        mesh=mesh,
        out_type=jax.ShapeDtypeStruct((B * M,), jnp.int32),
        compiler_params=pltpu.CompilerParams(needs_layout_passes=False),
        scratch_types=[
            pltpu.VMEM((L,), jnp.uint32),
            pltpu.VMEM((M,), jnp.int32),
            pltpu.VMEM((256,), jnp.int32),
        ],
    )
    def _topk(keys_hbm, out_hbm, keys_v, idx_v, hist_v):
        wid = lax.axis_index("s") * 2 + lax.axis_index("c")

        @pl.when(wid < B)
        def _():
            pltpu.sync_copy(keys_hbm.at[pl.ds(wid * L, L)], keys_v)

            iota16 = lax.broadcasted_iota(jnp.int32, (16,), 0)
            ones16 = jnp.full((16,), 1, jnp.int32)
            zero16 = jnp.zeros((16,), jnp.int32)
            fifteen = jnp.full((16,), 15, jnp.int32)
            gdn = lax.GatherDimensionNumbers(
                offset_dims=(), collapsed_slice_dims=(0,), start_index_map=(0,))

            def gat(x, idxv):
                return lax.gather(x, idxv[:, None], gdn, (1,),
                                  mode=lax.GatherScatterMode.PROMISE_IN_BOUNDS)

            def splat_last(x):
                return gat(x, fifteen)

            def rev(x):
                return lax.rev(x, (0,))

            prefix = jnp.zeros((16,), jnp.uint32)
            rem = jnp.full((16,), M, jnp.int32)
            for r in range(4):
                bsh = jnp.uint32(24 - 8 * r)

                def hz(i, c):
                    hist_v[pl.ds(i * 16, 16)] = zero16
                    return c

                lax.fori_loop(0, 16, hz, 0, unroll=4)

                def hb(i, c, _bsh=bsh, _prefix=prefix, _r=r):
                    kv = keys_v[pl.ds(i * 16, 16)]
                    bucket = ((kv >> _bsh) & jnp.uint32(255)).astype(jnp.int32)
                    if _r == 0:
                        plsc.addupdate_scatter(hist_v, [bucket], ones16)
                    else:
                        match = (kv >> (_bsh + 8)) == (_prefix >> (_bsh + 8))
                        plsc.addupdate_scatter(hist_v, [bucket], ones16,
                                               mask=match)
                    return c

                lax.fori_loop(0, nv, hb, 0, unroll=4)

                def hs(c, carry, _rem=rem):
                    cum, bfound, g = carry
                    cc = 15 - c
                    h = hist_v[pl.ds(cc * 16, 16)]
                    suf_in = rev(plsc.cumsum(rev(h)))
                    suffix = suf_in + cum
                    cond = suffix >= _rem
                    npos = plsc.all_reduce_population_count(cond)
                    hit = jnp.logical_and(npos > 0, bfound < 0)
                    l = npos - 1
                    gg = jnp.where(l >= 15, cum,
                                   gat(suffix, jnp.minimum(l + 1, fifteen)))
                    bfound = jnp.where(hit, cc * 16 + l, bfound)
                    g = jnp.where(hit, gg, g)
                    cum = gat(suffix, zero16)
                    return cum, bfound, g

                _, b, g = lax.fori_loop(
                    0, 16, hs,
                    (zero16, jnp.full((16,), -1, jnp.int32), zero16))
                prefix = prefix | (b.astype(jnp.uint32) << bsh)
                rem = rem - g

            thr = prefix
            k_eq = rem

            def sel(i, carry):
                off, eqb = carry
                kv = keys_v[pl.ds(i * 16, 16)]
                gt = kv > thr
                eq = kv == thr
                eqc = plsc.cumsum(jnp.where(eq, jnp.int32(1), jnp.int32(0)))
                keep = jnp.logical_and(eq, eqb + eqc - 1 < k_eq)
                m = jnp.logical_or(gt, keep)
                mi = jnp.where(m, jnp.int32(1), jnp.int32(0))
                incl = plsc.cumsum(mi)
                pos = off + incl - 1
                plsc.store_scatter(idx_v, [pos], iota16 + i * 16, mask=m)
                return off + splat_last(incl), eqb + splat_last(eqc)

            lax.fori_loop(0, nv, sel, (zero16, zero16), unroll=4)
            pltpu.sync_copy(idx_v, out_hbm.at[pl.ds(wid * M, M)])

    return _topk


def kernel(enc, W_q, W_k):
    B, L, d = enc.shape
    M = L // 4
    probs, keys = _compute_probs(enc, W_q, W_k)
    idx = _make_topk(B, L, M)(keys).reshape(B, M)
    return probs, idx, probs
